# baseline (speedup 1.0000x reference)
import functools

import jax
import jax.numpy as jnp
from jax import lax
from jax.experimental import pallas as pl
from jax.experimental.pallas import tpu as pltpu
from jax.experimental.pallas import tpu_sc as plsc


def _shuffle_block(buf_in, buf_out, nrows):
    iota = lax.broadcasted_iota(jnp.int32, (16,), 0)

    def kbody(k4, carry):
        for ku in range(4):
            k = k4 * 4 + ku
            perm = jnp.bitwise_and(iota + k, 15)
            st0 = perm * 32 + iota
            for rb in range(0, nrows, 16):
                for c0 in (0, 16):
                    v = plsc.load_gather(buf_in, [iota + c0, perm + rb])
                    plsc.store_scatter(buf_out, [st0 + (rb * 32 + c0)], v)
        return carry

    lax.fori_loop(0, 4, kbody, 0)


def _make_transpose(v, d):
    info = plsc.get_sparse_core_info()
    nc, ns = info.num_cores, info.num_subcores
    nw = nc * ns
    assert d == 32
    blk_r = 128
    nbuf = 4
    nfull = v // blk_r
    tail = v % blk_r
    per_w = nfull // nw
    extra = nfull % nw
    assert per_w % nbuf == 0 and per_w >= 2 * nbuf

    mesh = plsc.VectorSubcoreMesh(core_axis_name="c", subcore_axis_name="s")

    @functools.partial(
        pl.kernel,
        mesh=mesh,
        compiler_params=pltpu.CompilerParams(needs_layout_passes=False),
        out_type=jax.ShapeDtypeStruct((v * d,), jnp.float32),
        scratch_types=(
            [pltpu.VMEM((d, blk_r), jnp.float32)] * nbuf
            + [pltpu.VMEM((blk_r * d,), jnp.float32)] * nbuf
            + [pltpu.VMEM((d, tail if tail else 1), jnp.float32)]
            + [pltpu.SemaphoreType.DMA] * (2 * nbuf)
        ),
    )
    def transpose_kernel(tt_hbm, tp_hbm, *scratch):
        bins = list(scratch[:nbuf])
        bouts = list(scratch[nbuf:2 * nbuf])
        btail = scratch[2 * nbuf]
        isems = list(scratch[2 * nbuf + 1:3 * nbuf + 1])
        osems = list(scratch[3 * nbuf + 1:4 * nbuf + 1])
        wid = lax.axis_index("s") * nc + lax.axis_index("c")

        def in_slice(blk):
            return tt_hbm.at[:, pl.ds(pl.multiple_of(blk * blk_r, 128),
                                      blk_r)]

        def out_slice(blk):
            return tp_hbm.at[pl.ds(pl.multiple_of(blk * (blk_r * d), 8),
                                   blk_r * d)]

        def start_in(blk, p):
            pltpu.async_copy(in_slice(blk), bins[p], isems[p])

        def wait_in(p):
            pltpu.make_async_copy(in_slice(0), bins[p], isems[p]).wait()

        def start_out(blk, p):
            pltpu.async_copy(bouts[p], out_slice(blk), osems[p])

        def wait_out(p):
            pltpu.make_async_copy(bouts[p], out_slice(0), osems[p]).wait()

        for j in range(nbuf - 1):
            start_in(wid + nw * j, j)

        def slot(j, p, first_round):
            wait_in(p)
            nblk = wid + nw * (j + nbuf - 1)

            @pl.when(nblk < nfull)
            def _():
                start_in(nblk, (p + nbuf - 1) % nbuf)

            if not first_round:
                wait_out(p)
            _shuffle_block(bins[p], bouts[p], blk_r)
            start_out(wid + nw * j, p)

        for j in range(nbuf):
            slot(j, j, True)

        def body(i2, carry):
            for k in range(nbuf):
                slot(i2 * nbuf + k, k, False)
            return carry

        lax.fori_loop(1, per_w // nbuf, body, 0)
        for q in range(nbuf):
            wait_out(q)

        if extra:
            ep = per_w % nbuf

            @pl.when(wid < extra)
            def _():
                wait_in(ep)
                _shuffle_block(bins[ep], bouts[ep], blk_r)
                pltpu.sync_copy(bouts[ep], out_slice(wid + nw * per_w))

        if tail:
            @pl.when(wid == extra)
            def _():
                pltpu.sync_copy(tt_hbm.at[:, pl.ds(nfull * blk_r, tail)],
                                btail)
                _shuffle_block(btail, bouts[0], tail)
                pltpu.sync_copy(
                    bouts[0].at[pl.ds(0, tail * d)],
                    tp_hbm.at[pl.ds(nfull * (blk_r * d), tail * d)])

    return transpose_kernel


def _make_gather(n, v, d):
    info = plsc.get_sparse_core_info()
    nc, ns = info.num_cores, info.num_subcores
    nw = nc * ns
    assert n % nw == 0
    b_per_w = n // nw
    chunk = 1664
    while b_per_w % chunk != 0:
        chunk //= 2
    nchunks = b_per_w // chunk

    mesh = plsc.VectorSubcoreMesh(core_axis_name="c", subcore_axis_name="s")

    @functools.partial(
        pl.kernel,
        mesh=mesh,
        compiler_params=pltpu.CompilerParams(use_tc_tiling_on_sc=False),
        out_type=jax.ShapeDtypeStruct((n, d), jnp.float32),
        scratch_types=[
            pltpu.VMEM((b_per_w,), jnp.int32),
            pltpu.VMEM((chunk, d), jnp.float32),
            pltpu.VMEM((chunk, d), jnp.float32),
            pltpu.SemaphoreType.DMA,
            pltpu.SemaphoreType.DMA,
            pltpu.SemaphoreType.DMA,
            pltpu.SemaphoreType.DMA,
        ],
    )
    def gather_kernel(table_hbm, idx_hbm, out_hbm, idx_v,
                      rows0, rows1, gsem0, gsem1, osem0, osem1):
        wid = lax.axis_index("s") * nc + lax.axis_index("c")
        base = wid * b_per_w
        pltpu.sync_copy(idx_hbm.at[pl.ds(base, b_per_w)], idx_v)

        rows = [rows0, rows1]
        gsems = [gsem0, gsem1]
        osems = [osem0, osem1]
        g_desc = [None, None]
        o_desc = [None, None]

        def issue_gather(g):
            bb = g % 2
            g_desc[bb] = pltpu.async_copy(
                table_hbm.at[idx_v.at[pl.ds(g * chunk, chunk)]],
                rows[bb], gsems[bb])

        def issue_out(g):
            bb = g % 2
            o_desc[bb] = pltpu.async_copy(
                rows[bb], out_hbm.at[pl.ds(base + g * chunk, chunk)],
                osems[bb])

        issue_gather(0)
        for g in range(nchunks):
            bb = g % 2
            g_desc[bb].wait()
            if g >= 1:
                o_desc[1 - bb].wait()
            if g + 1 < nchunks:
                issue_gather(g + 1)
            issue_out(g)
        o_desc[(nchunks - 1) % 2].wait()

    return gather_kernel


def _make_relayout(bsz, fno, d):
    info = plsc.get_sparse_core_info()
    nc, ns = info.num_cores, info.num_subcores
    nw = nc * ns
    ngrp = fno * (bsz // 128)
    assert bsz % 128 == 0 and ngrp % nw == 0 and d == 32
    per_w = ngrp // nw
    nbuf = 4
    assert per_w % nbuf == 0 and per_w >= 2 * nbuf
    nbb = bsz // 128

    mesh = plsc.VectorSubcoreMesh(core_axis_name="c", subcore_axis_name="s")

    @functools.partial(
        pl.kernel,
        mesh=mesh,
        compiler_params=pltpu.CompilerParams(needs_layout_passes=False),
        out_type=jax.ShapeDtypeStruct((fno, d, bsz), jnp.float32),
        scratch_types=(
            [pltpu.VMEM((128 * d,), jnp.float32)] * nbuf
            + [pltpu.VMEM((d, 128), jnp.float32)] * nbuf
            + [pltpu.SemaphoreType.DMA] * (2 * nbuf)
        ),
    )
    def relayout_kernel(x_hbm, o_hbm, *scratch):
        bins = list(scratch[:nbuf])
        bouts = list(scratch[nbuf:2 * nbuf])
        isems = list(scratch[2 * nbuf:3 * nbuf])
        osems = list(scratch[3 * nbuf:4 * nbuf])
        wid = lax.axis_index("s") * nc + lax.axis_index("c")
        iota = lax.broadcasted_iota(jnp.int32, (16,), 0)

        def in_slice(g):
            return x_hbm.at[pl.ds(pl.multiple_of(g * (128 * d), 8),
                                  128 * d)]

        def out_slice(g):
            f = g // nbb
            b0 = pl.multiple_of((g % nbb) * 128, 128)
            return o_hbm.at[f, :, pl.ds(b0, 128)]

        def start_in(g, p):
            pltpu.async_copy(in_slice(g), bins[p], isems[p])

        def wait_in(p):
            pltpu.make_async_copy(in_slice(0), bins[p], isems[p]).wait()

        def wait_out(p):
            pltpu.make_async_copy(
                bouts[p], o_hbm.at[0, :, pl.ds(0, 128)], osems[p]).wait()

        def shuffle(p):
            def kbody(k4, carry):
                for ku in range(4):
                    k = k4 * 4 + ku
                    perm = jnp.bitwise_and(iota + k, 15)
                    for b0 in range(0, 128, 16):
                        for c0 in (0, 16):
                            vv = plsc.load_gather(
                                bins[p], [iota * d + perm + (b0 * d + c0)])
                            plsc.store_scatter(
                                bouts[p], [perm + c0, iota + b0], vv)
                return carry

            lax.fori_loop(0, 4, kbody, 0)

        for j in range(nbuf - 1):
            start_in(wid + nw * j, j)

        def slot(j, p, first_round):
            wait_in(p)
            pnext = (p + nbuf - 1) % nbuf
            if isinstance(j, int):
                if j + nbuf - 1 < per_w:
                    start_in(wid + nw * (j + nbuf - 1), pnext)
            else:
                @pl.when(j + nbuf - 1 < per_w)
                def _():
                    start_in(wid + nw * (j + nbuf - 1), pnext)

            if not first_round:
                wait_out(p)
            shuffle(p)
            pltpu.async_copy(bouts[p], out_slice(wid + nw * j), osems[p])

        for j in range(nbuf):
            slot(j, j, True)

        def body(i2, carry):
            for k in range(nbuf):
                slot(i2 * nbuf + k, k, False)
            return carry

        lax.fori_loop(1, per_w // nbuf, body, 0)
        for q in range(nbuf):
            wait_out(q)

    return relayout_kernel


def kernel(inputs, table):
    b, f = inputs.shape
    v, d = table.shape
    n = b * f
    flat_idx = inputs.T.reshape(n).astype(jnp.int32)
    tp = _make_transpose(v, d)(table.T).reshape(v, d)
    out = _make_gather(n, v, d)(tp, flat_idx)
    o_t = _make_relayout(b, f, d)(out.reshape(n * d))
    return o_t.transpose(2, 0, 1)


# speedup vs baseline: 1.2995x; 1.2995x over previous
import functools

import jax
import jax.numpy as jnp
from jax import lax
from jax.experimental import pallas as pl
from jax.experimental.pallas import tpu as pltpu
from jax.experimental.pallas import tpu_sc as plsc


def _shuffle_block(buf_in, buf_out, nrows):
    iota = lax.broadcasted_iota(jnp.int32, (16,), 0)

    def kbody(k, carry):
        perm = jnp.bitwise_and(iota + k, 15)
        st0 = perm * 32 + iota
        for rb in range(0, nrows, 16):
            for c0 in (0, 16):
                v = plsc.load_gather(buf_in, [iota + c0, perm + rb])
                plsc.store_scatter(buf_out, [st0 + (rb * 32 + c0)], v)
        return carry

    lax.fori_loop(0, 16, kbody, 0)


def _make_transpose(v, d):
    info = plsc.get_sparse_core_info()
    nc, ns = info.num_cores, info.num_subcores
    nw = nc * ns
    assert d == 32
    blk_r = 128
    nbuf = 4
    nfull = v // blk_r
    tail = v % blk_r
    per_w = nfull // nw
    extra = nfull % nw
    assert per_w % nbuf == 0 and per_w >= 2 * nbuf

    mesh = plsc.VectorSubcoreMesh(core_axis_name="c", subcore_axis_name="s")

    @functools.partial(
        pl.kernel,
        mesh=mesh,
        compiler_params=pltpu.CompilerParams(needs_layout_passes=False),
        out_type=jax.ShapeDtypeStruct((v * d,), jnp.float32),
        scratch_types=(
            [pltpu.VMEM((d, blk_r), jnp.float32)] * nbuf
            + [pltpu.VMEM((blk_r * d,), jnp.float32)] * nbuf
            + [pltpu.VMEM((d, tail if tail else 1), jnp.float32)]
            + [pltpu.SemaphoreType.DMA] * (2 * nbuf)
        ),
    )
    def transpose_kernel(tt_hbm, tp_hbm, *scratch):
        bins = list(scratch[:nbuf])
        bouts = list(scratch[nbuf:2 * nbuf])
        btail = scratch[2 * nbuf]
        isems = list(scratch[2 * nbuf + 1:3 * nbuf + 1])
        osems = list(scratch[3 * nbuf + 1:4 * nbuf + 1])
        wid = lax.axis_index("s") * nc + lax.axis_index("c")

        def in_slice(blk):
            return tt_hbm.at[:, pl.ds(pl.multiple_of(blk * blk_r, 128),
                                      blk_r)]

        def out_slice(blk):
            return tp_hbm.at[pl.ds(pl.multiple_of(blk * (blk_r * d), 8),
                                   blk_r * d)]

        def start_in(blk, p):
            pltpu.async_copy(in_slice(blk), bins[p], isems[p])

        def wait_in(p):
            pltpu.make_async_copy(in_slice(0), bins[p], isems[p]).wait()

        def start_out(blk, p):
            pltpu.async_copy(bouts[p], out_slice(blk), osems[p])

        def wait_out(p):
            pltpu.make_async_copy(bouts[p], out_slice(0), osems[p]).wait()

        for j in range(nbuf - 1):
            start_in(wid + nw * j, j)

        def slot(j, p, first_round):
            wait_in(p)
            nblk = wid + nw * (j + nbuf - 1)

            @pl.when(nblk < nfull)
            def _():
                start_in(nblk, (p + nbuf - 1) % nbuf)

            if not first_round:
                wait_out(p)
            _shuffle_block(bins[p], bouts[p], blk_r)
            start_out(wid + nw * j, p)

        for j in range(nbuf):
            slot(j, j, True)

        def body(i2, carry):
            for k in range(nbuf):
                slot(i2 * nbuf + k, k, False)
            return carry

        lax.fori_loop(1, per_w // nbuf, body, 0)
        for q in range(nbuf):
            wait_out(q)

        if extra:
            ep = per_w % nbuf

            @pl.when(wid < extra)
            def _():
                wait_in(ep)
                _shuffle_block(bins[ep], bouts[ep], blk_r)
                pltpu.sync_copy(bouts[ep], out_slice(wid + nw * per_w))

        if tail:
            @pl.when(wid == extra)
            def _():
                pltpu.sync_copy(tt_hbm.at[:, pl.ds(nfull * blk_r, tail)],
                                btail)
                _shuffle_block(btail, bouts[0], tail)
                pltpu.sync_copy(
                    bouts[0].at[pl.ds(0, tail * d)],
                    tp_hbm.at[pl.ds(nfull * (blk_r * d), tail * d)])

    return transpose_kernel


def _make_gather(n, v, d):
    info = plsc.get_sparse_core_info()
    nc, ns = info.num_cores, info.num_subcores
    nw = nc * ns
    assert n % nw == 0
    b_per_w = n // nw
    chunk = 1664
    while b_per_w % chunk != 0:
        chunk //= 2
    nchunks = b_per_w // chunk

    mesh = plsc.VectorSubcoreMesh(core_axis_name="c", subcore_axis_name="s")

    @functools.partial(
        pl.kernel,
        mesh=mesh,
        compiler_params=pltpu.CompilerParams(use_tc_tiling_on_sc=False),
        out_type=jax.ShapeDtypeStruct((n, d), jnp.float32),
        scratch_types=[
            pltpu.VMEM((b_per_w,), jnp.int32),
            pltpu.VMEM((chunk, d), jnp.float32),
            pltpu.VMEM((chunk, d), jnp.float32),
            pltpu.SemaphoreType.DMA,
            pltpu.SemaphoreType.DMA,
            pltpu.SemaphoreType.DMA,
            pltpu.SemaphoreType.DMA,
        ],
    )
    def gather_kernel(table_hbm, idx_hbm, out_hbm, idx_v,
                      rows0, rows1, gsem0, gsem1, osem0, osem1):
        wid = lax.axis_index("s") * nc + lax.axis_index("c")
        base = wid * b_per_w
        pltpu.sync_copy(idx_hbm.at[pl.ds(base, b_per_w)], idx_v)

        rows = [rows0, rows1]
        gsems = [gsem0, gsem1]
        osems = [osem0, osem1]
        g_desc = [None, None]
        o_desc = [None, None]

        def issue_gather(g):
            bb = g % 2
            g_desc[bb] = pltpu.async_copy(
                table_hbm.at[idx_v.at[pl.ds(g * chunk, chunk)]],
                rows[bb], gsems[bb])

        def issue_out(g):
            bb = g % 2
            o_desc[bb] = pltpu.async_copy(
                rows[bb], out_hbm.at[pl.ds(base + g * chunk, chunk)],
                osems[bb])

        issue_gather(0)
        for g in range(nchunks):
            bb = g % 2
            g_desc[bb].wait()
            if g >= 1:
                o_desc[1 - bb].wait()
            if g + 1 < nchunks:
                issue_gather(g + 1)
            issue_out(g)
        o_desc[(nchunks - 1) % 2].wait()

    return gather_kernel


def _make_relayout(bsz, fno, d):
    info = plsc.get_sparse_core_info()
    nc, ns = info.num_cores, info.num_subcores
    nw = nc * ns
    ngrp = fno * (bsz // 128)
    assert bsz % 128 == 0 and ngrp % nw == 0 and d == 32
    per_w = ngrp // nw
    nbuf = 4
    assert per_w % nbuf == 0 and per_w >= 2 * nbuf
    nbb = bsz // 128

    mesh = plsc.VectorSubcoreMesh(core_axis_name="c", subcore_axis_name="s")

    @functools.partial(
        pl.kernel,
        mesh=mesh,
        compiler_params=pltpu.CompilerParams(needs_layout_passes=False),
        out_type=jax.ShapeDtypeStruct((fno, d, bsz), jnp.float32),
        scratch_types=(
            [pltpu.VMEM((128 * d,), jnp.float32)] * nbuf
            + [pltpu.VMEM((d, 128), jnp.float32)] * nbuf
            + [pltpu.SemaphoreType.DMA] * (2 * nbuf)
        ),
    )
    def relayout_kernel(x_hbm, o_hbm, *scratch):
        bins = list(scratch[:nbuf])
        bouts = list(scratch[nbuf:2 * nbuf])
        isems = list(scratch[2 * nbuf:3 * nbuf])
        osems = list(scratch[3 * nbuf:4 * nbuf])
        wid = lax.axis_index("s") * nc + lax.axis_index("c")
        iota = lax.broadcasted_iota(jnp.int32, (16,), 0)

        def in_slice(g):
            return x_hbm.at[pl.ds(pl.multiple_of(g * (128 * d), 8),
                                  128 * d)]

        def out_slice(g):
            f = g // nbb
            b0 = pl.multiple_of((g % nbb) * 128, 128)
            return o_hbm.at[f, :, pl.ds(b0, 128)]

        def start_in(g, p):
            pltpu.async_copy(in_slice(g), bins[p], isems[p])

        def wait_in(p):
            pltpu.make_async_copy(in_slice(0), bins[p], isems[p]).wait()

        def wait_out(p):
            pltpu.make_async_copy(
                bouts[p], o_hbm.at[0, :, pl.ds(0, 128)], osems[p]).wait()

        def shuffle(p):
            def kbody(k, carry):
                perm = jnp.bitwise_and(iota + k, 15)
                for b0 in range(0, 128, 16):
                    for c0 in (0, 16):
                        vv = plsc.load_gather(
                            bins[p], [iota * d + perm + (b0 * d + c0)])
                        plsc.store_scatter(
                            bouts[p], [perm + c0, iota + b0], vv)
                return carry

            lax.fori_loop(0, 16, kbody, 0)

        for j in range(nbuf - 1):
            start_in(wid + nw * j, j)

        def slot(j, p, first_round):
            wait_in(p)
            pnext = (p + nbuf - 1) % nbuf
            if isinstance(j, int):
                if j + nbuf - 1 < per_w:
                    start_in(wid + nw * (j + nbuf - 1), pnext)
            else:
                @pl.when(j + nbuf - 1 < per_w)
                def _():
                    start_in(wid + nw * (j + nbuf - 1), pnext)

            if not first_round:
                wait_out(p)
            shuffle(p)
            pltpu.async_copy(bouts[p], out_slice(wid + nw * j), osems[p])

        for j in range(nbuf):
            slot(j, j, True)

        def body(i2, carry):
            for k in range(nbuf):
                slot(i2 * nbuf + k, k, False)
            return carry

        lax.fori_loop(1, per_w // nbuf, body, 0)
        for q in range(nbuf):
            wait_out(q)

    return relayout_kernel


def kernel(inputs, table):
    b, f = inputs.shape
    v, d = table.shape
    n = b * f
    flat_idx = inputs.T.reshape(n).astype(jnp.int32)
    tp = _make_transpose(v, d)(table.T).reshape(v, d)
    out = _make_gather(n, v, d)(tp, flat_idx)
    o_t = _make_relayout(b, f, d)(out.reshape(n * d))
    return o_t.transpose(2, 0, 1)


# speedup vs baseline: 2.4697x; 1.9005x over previous
import functools

import jax
import jax.numpy as jnp
from jax import lax
from jax.experimental import pallas as pl
from jax.experimental.pallas import tpu as pltpu
from jax.experimental.pallas import tpu_sc as plsc


def _shuffle_block(buf_in, buf_out, nrows):
    iota = lax.broadcasted_iota(jnp.int32, (16,), 0)

    @plsc.parallel_loop(0, 16)
    def kbody(k):
        perm = jnp.bitwise_and(iota + k, 15)
        st0 = perm * 32 + iota
        for rb in range(0, nrows, 16):
            for c0 in (0, 16):
                v = plsc.load_gather(buf_in, [iota + c0, perm + rb])
                plsc.store_scatter(buf_out, [st0 + (rb * 32 + c0)], v)


def _make_transpose(v, d):
    info = plsc.get_sparse_core_info()
    nc, ns = info.num_cores, info.num_subcores
    nw = nc * ns
    assert d == 32
    blk_r = 128
    nbuf = 4
    nfull = v // blk_r
    tail = v % blk_r
    per_w = nfull // nw
    extra = nfull % nw
    assert per_w % nbuf == 0 and per_w >= 2 * nbuf

    mesh = plsc.VectorSubcoreMesh(core_axis_name="c", subcore_axis_name="s")

    @functools.partial(
        pl.kernel,
        mesh=mesh,
        compiler_params=pltpu.CompilerParams(needs_layout_passes=False),
        out_type=jax.ShapeDtypeStruct((v * d,), jnp.float32),
        scratch_types=(
            [pltpu.VMEM((d, blk_r), jnp.float32)] * nbuf
            + [pltpu.VMEM((blk_r * d,), jnp.float32)] * nbuf
            + [pltpu.VMEM((d, tail if tail else 1), jnp.float32)]
            + [pltpu.SemaphoreType.DMA] * (2 * nbuf)
        ),
    )
    def transpose_kernel(tt_hbm, tp_hbm, *scratch):
        bins = list(scratch[:nbuf])
        bouts = list(scratch[nbuf:2 * nbuf])
        btail = scratch[2 * nbuf]
        isems = list(scratch[2 * nbuf + 1:3 * nbuf + 1])
        osems = list(scratch[3 * nbuf + 1:4 * nbuf + 1])
        wid = lax.axis_index("s") * nc + lax.axis_index("c")

        def in_slice(blk):
            return tt_hbm.at[:, pl.ds(pl.multiple_of(blk * blk_r, 128),
                                      blk_r)]

        def out_slice(blk):
            return tp_hbm.at[pl.ds(pl.multiple_of(blk * (blk_r * d), 8),
                                   blk_r * d)]

        def start_in(blk, p):
            pltpu.async_copy(in_slice(blk), bins[p], isems[p])

        def wait_in(p):
            pltpu.make_async_copy(in_slice(0), bins[p], isems[p]).wait()

        def start_out(blk, p):
            pltpu.async_copy(bouts[p], out_slice(blk), osems[p])

        def wait_out(p):
            pltpu.make_async_copy(bouts[p], out_slice(0), osems[p]).wait()

        for j in range(nbuf - 1):
            start_in(wid + nw * j, j)

        def slot(j, p, first_round):
            wait_in(p)
            nblk = wid + nw * (j + nbuf - 1)

            @pl.when(nblk < nfull)
            def _():
                start_in(nblk, (p + nbuf - 1) % nbuf)

            if not first_round:
                wait_out(p)
            _shuffle_block(bins[p], bouts[p], blk_r)
            start_out(wid + nw * j, p)

        for j in range(nbuf):
            slot(j, j, True)

        def body(i2, carry):
            for k in range(nbuf):
                slot(i2 * nbuf + k, k, False)
            return carry

        lax.fori_loop(1, per_w // nbuf, body, 0)
        for q in range(nbuf):
            wait_out(q)

        if extra:
            ep = per_w % nbuf

            @pl.when(wid < extra)
            def _():
                wait_in(ep)
                _shuffle_block(bins[ep], bouts[ep], blk_r)
                pltpu.sync_copy(bouts[ep], out_slice(wid + nw * per_w))

        if tail:
            @pl.when(wid == extra)
            def _():
                pltpu.sync_copy(tt_hbm.at[:, pl.ds(nfull * blk_r, tail)],
                                btail)
                _shuffle_block(btail, bouts[0], tail)
                pltpu.sync_copy(
                    bouts[0].at[pl.ds(0, tail * d)],
                    tp_hbm.at[pl.ds(nfull * (blk_r * d), tail * d)])

    return transpose_kernel


def _make_gather(n, v, d):
    info = plsc.get_sparse_core_info()
    nc, ns = info.num_cores, info.num_subcores
    nw = nc * ns
    assert n % nw == 0
    b_per_w = n // nw
    chunk = 1664
    while b_per_w % chunk != 0:
        chunk //= 2
    nchunks = b_per_w // chunk

    mesh = plsc.VectorSubcoreMesh(core_axis_name="c", subcore_axis_name="s")

    @functools.partial(
        pl.kernel,
        mesh=mesh,
        compiler_params=pltpu.CompilerParams(use_tc_tiling_on_sc=False),
        out_type=jax.ShapeDtypeStruct((n, d), jnp.float32),
        scratch_types=[
            pltpu.VMEM((b_per_w,), jnp.int32),
            pltpu.VMEM((chunk, d), jnp.float32),
            pltpu.VMEM((chunk, d), jnp.float32),
            pltpu.SemaphoreType.DMA,
            pltpu.SemaphoreType.DMA,
            pltpu.SemaphoreType.DMA,
            pltpu.SemaphoreType.DMA,
        ],
    )
    def gather_kernel(table_hbm, idx_hbm, out_hbm, idx_v,
                      rows0, rows1, gsem0, gsem1, osem0, osem1):
        wid = lax.axis_index("s") * nc + lax.axis_index("c")
        base = wid * b_per_w
        pltpu.sync_copy(idx_hbm.at[pl.ds(base, b_per_w)], idx_v)

        rows = [rows0, rows1]
        gsems = [gsem0, gsem1]
        osems = [osem0, osem1]
        g_desc = [None, None]
        o_desc = [None, None]

        def issue_gather(g):
            bb = g % 2
            g_desc[bb] = pltpu.async_copy(
                table_hbm.at[idx_v.at[pl.ds(g * chunk, chunk)]],
                rows[bb], gsems[bb])

        def issue_out(g):
            bb = g % 2
            o_desc[bb] = pltpu.async_copy(
                rows[bb], out_hbm.at[pl.ds(base + g * chunk, chunk)],
                osems[bb])

        issue_gather(0)
        for g in range(nchunks):
            bb = g % 2
            g_desc[bb].wait()
            if g >= 1:
                o_desc[1 - bb].wait()
            if g + 1 < nchunks:
                issue_gather(g + 1)
            issue_out(g)
        o_desc[(nchunks - 1) % 2].wait()

    return gather_kernel


def _make_relayout(bsz, fno, d):
    info = plsc.get_sparse_core_info()
    nc, ns = info.num_cores, info.num_subcores
    nw = nc * ns
    ngrp = fno * (bsz // 128)
    assert bsz % 128 == 0 and ngrp % nw == 0 and d == 32
    per_w = ngrp // nw
    nbuf = 4
    assert per_w % nbuf == 0 and per_w >= 2 * nbuf
    nbb = bsz // 128

    mesh = plsc.VectorSubcoreMesh(core_axis_name="c", subcore_axis_name="s")

    @functools.partial(
        pl.kernel,
        mesh=mesh,
        compiler_params=pltpu.CompilerParams(needs_layout_passes=False),
        out_type=jax.ShapeDtypeStruct((fno, d, bsz), jnp.float32),
        scratch_types=(
            [pltpu.VMEM((128 * d,), jnp.float32)] * nbuf
            + [pltpu.VMEM((d, 128), jnp.float32)] * nbuf
            + [pltpu.SemaphoreType.DMA] * (2 * nbuf)
        ),
    )
    def relayout_kernel(x_hbm, o_hbm, *scratch):
        bins = list(scratch[:nbuf])
        bouts = list(scratch[nbuf:2 * nbuf])
        isems = list(scratch[2 * nbuf:3 * nbuf])
        osems = list(scratch[3 * nbuf:4 * nbuf])
        wid = lax.axis_index("s") * nc + lax.axis_index("c")
        iota = lax.broadcasted_iota(jnp.int32, (16,), 0)

        def in_slice(g):
            return x_hbm.at[pl.ds(pl.multiple_of(g * (128 * d), 8),
                                  128 * d)]

        def out_slice(g):
            f = g // nbb
            b0 = pl.multiple_of((g % nbb) * 128, 128)
            return o_hbm.at[f, :, pl.ds(b0, 128)]

        def start_in(g, p):
            pltpu.async_copy(in_slice(g), bins[p], isems[p])

        def wait_in(p):
            pltpu.make_async_copy(in_slice(0), bins[p], isems[p]).wait()

        def wait_out(p):
            pltpu.make_async_copy(
                bouts[p], o_hbm.at[0, :, pl.ds(0, 128)], osems[p]).wait()

        def shuffle(p):
            @plsc.parallel_loop(0, 16)
            def kbody(k):
                perm = jnp.bitwise_and(iota + k, 15)
                for b0 in range(0, 128, 16):
                    for c0 in (0, 16):
                        vv = plsc.load_gather(
                            bins[p], [iota * d + perm + (b0 * d + c0)])
                        plsc.store_scatter(
                            bouts[p], [perm + c0, iota + b0], vv)

        for j in range(nbuf - 1):
            start_in(wid + nw * j, j)

        def slot(j, p, first_round):
            wait_in(p)
            pnext = (p + nbuf - 1) % nbuf
            if isinstance(j, int):
                if j + nbuf - 1 < per_w:
                    start_in(wid + nw * (j + nbuf - 1), pnext)
            else:
                @pl.when(j + nbuf - 1 < per_w)
                def _():
                    start_in(wid + nw * (j + nbuf - 1), pnext)

            if not first_round:
                wait_out(p)
            shuffle(p)
            pltpu.async_copy(bouts[p], out_slice(wid + nw * j), osems[p])

        for j in range(nbuf):
            slot(j, j, True)

        def body(i2, carry):
            for k in range(nbuf):
                slot(i2 * nbuf + k, k, False)
            return carry

        lax.fori_loop(1, per_w // nbuf, body, 0)
        for q in range(nbuf):
            wait_out(q)

    return relayout_kernel


def kernel(inputs, table):
    b, f = inputs.shape
    v, d = table.shape
    n = b * f
    flat_idx = inputs.T.reshape(n).astype(jnp.int32)
    tp = _make_transpose(v, d)(table.T).reshape(v, d)
    out = _make_gather(n, v, d)(tp, flat_idx)
    o_t = _make_relayout(b, f, d)(out.reshape(n * d))
    return o_t.transpose(2, 0, 1)
